# baseline (speedup 1.0000x reference)
import functools

import jax
import jax.numpy as jnp
from jax.experimental import pallas as pl
from jax.experimental.pallas import tpu as pltpu

_N = 1000
_K = 12544
_MID = 1024
_KB = 1792


def _fused_mlp(x_ref, w1_ref, b1_ref, w2_ref, b2_ref, wc_ref, bc_ref,
               wb_ref, bb_ref, score_ref, bbox_ref, acc_ref):
    k = pl.program_id(0)

    @pl.when(k == 0)
    def _init():
        acc_ref[...] = jnp.zeros_like(acc_ref)

    xb = x_ref[...].astype(jnp.bfloat16)
    w1b = w1_ref[...].astype(jnp.bfloat16)
    acc_ref[...] += jnp.dot(xb, w1b, preferred_element_type=jnp.float32)

    @pl.when(k == pl.num_programs(0) - 1)
    def _epilogue():
        h1 = jnp.maximum(acc_ref[...] + b1_ref[...], 0.0)
        h2 = jnp.maximum(
            jnp.dot(h1.astype(jnp.bfloat16), w2_ref[...].astype(jnp.bfloat16),
                    preferred_element_type=jnp.float32) + b2_ref[...], 0.0)
        h2b = h2.astype(jnp.bfloat16)
        score_ref[...] = jnp.dot(h2b, wc_ref[...].astype(jnp.bfloat16),
                                 preferred_element_type=jnp.float32) + bc_ref[...]
        bbox_ref[...] = jnp.dot(h2b, wb_ref[...].astype(jnp.bfloat16),
                                preferred_element_type=jnp.float32) + bb_ref[...]


@functools.partial(jax.jit, static_argnums=())
def kernel(x, W1, b1, W2, b2, Wc, bc, Wb, bb):
    x = x.reshape(x.shape[0], -1)
    n = x.shape[0]
    nc = Wc.shape[1]
    nb = Wb.shape[1]
    steps = _K // _KB

    const = lambda k: (0, 0)
    score, bbox = pl.pallas_call(
        _fused_mlp,
        grid=(steps,),
        in_specs=[
            pl.BlockSpec((n, _KB), lambda k: (0, k)),
            pl.BlockSpec((_KB, _MID), lambda k: (k, 0)),
            pl.BlockSpec((1, _MID), const),
            pl.BlockSpec((_MID, _MID), const),
            pl.BlockSpec((1, _MID), const),
            pl.BlockSpec((_MID, nc), const),
            pl.BlockSpec((1, nc), const),
            pl.BlockSpec((_MID, nb), const),
            pl.BlockSpec((1, nb), const),
        ],
        out_specs=[
            pl.BlockSpec((n, nc), const),
            pl.BlockSpec((n, nb), const),
        ],
        out_shape=[
            jax.ShapeDtypeStruct((n, nc), jnp.float32),
            jax.ShapeDtypeStruct((n, nb), jnp.float32),
        ],
        scratch_shapes=[pltpu.VMEM((n, _MID), jnp.float32)],
        compiler_params=pltpu.CompilerParams(
            dimension_semantics=("arbitrary",),
        ),
    )(x, W1, b1.reshape(1, -1), W2, b2.reshape(1, -1),
      Wc, bc.reshape(1, -1), Wb, bb.reshape(1, -1))
    return (score, bbox)


# speedup vs baseline: 1.0041x; 1.0041x over previous
import functools

import jax
import jax.numpy as jnp
from jax.experimental import pallas as pl
from jax.experimental.pallas import tpu as pltpu

_N = 1000
_K = 12544
_MID = 1024
_KB = 1792


def _fused_mlp(x_ref, w1_ref, b1_ref, w2_ref, b2_ref, wc_ref, bc_ref,
               wb_ref, bb_ref, score_ref, bbox_ref, acc_ref):
    k = pl.program_id(0)

    @pl.when(k == 0)
    def _init():
        acc_ref[...] = jnp.zeros_like(acc_ref)

    acc_ref[...] += jnp.dot(x_ref[...], w1_ref[...],
                            preferred_element_type=jnp.float32)

    @pl.when(k == pl.num_programs(0) - 1)
    def _epilogue():
        h1 = jnp.maximum(acc_ref[...] + b1_ref[...], 0.0)
        h2 = jnp.maximum(
            jnp.dot(h1, w2_ref[...],
                    preferred_element_type=jnp.float32) + b2_ref[...], 0.0)
        score_ref[...] = jnp.dot(h2, wc_ref[...],
                                 preferred_element_type=jnp.float32) + bc_ref[...]
        bbox_ref[...] = jnp.dot(h2, wb_ref[...],
                                preferred_element_type=jnp.float32) + bb_ref[...]


@functools.partial(jax.jit, static_argnums=())
def kernel(x, W1, b1, W2, b2, Wc, bc, Wb, bb):
    x = x.reshape(x.shape[0], -1)
    n = x.shape[0]
    nc = Wc.shape[1]
    nb = Wb.shape[1]
    steps = _K // _KB

    const = lambda k: (0, 0)
    score, bbox = pl.pallas_call(
        _fused_mlp,
        grid=(steps,),
        in_specs=[
            pl.BlockSpec((n, _KB), lambda k: (0, k)),
            pl.BlockSpec((_KB, _MID), lambda k: (k, 0)),
            pl.BlockSpec((1, _MID), const),
            pl.BlockSpec((_MID, _MID), const),
            pl.BlockSpec((1, _MID), const),
            pl.BlockSpec((_MID, nc), const),
            pl.BlockSpec((1, nc), const),
            pl.BlockSpec((_MID, nb), const),
            pl.BlockSpec((1, nb), const),
        ],
        out_specs=[
            pl.BlockSpec((n, nc), const),
            pl.BlockSpec((n, nb), const),
        ],
        out_shape=[
            jax.ShapeDtypeStruct((n, nc), jnp.float32),
            jax.ShapeDtypeStruct((n, nb), jnp.float32),
        ],
        scratch_shapes=[pltpu.VMEM((n, _MID), jnp.float32)],
        compiler_params=pltpu.CompilerParams(
            dimension_semantics=("arbitrary",),
        ),
    )(x, W1, b1.reshape(1, -1), W2, b2.reshape(1, -1),
      Wc, bc.reshape(1, -1), Wb, bb.reshape(1, -1))
    return (score, bbox)
